# speedup vs baseline: 6.5198x; 6.5198x over previous
import functools

import jax
import jax.numpy as jnp
from jax.experimental import pallas as pl
from jax.experimental.pallas import tpu as pltpu
from jax.experimental.pallas import tpu_sc as plsc

_WINDOW = 128


@functools.partial(jax.jit, static_argnums=(2, 3))
def _sc_gather(table, idx_flat, num_indices, d_model):
    mesh = plsc.VectorSubcoreMesh(core_axis_name="core", subcore_axis_name="subcore")

    @functools.partial(
        pl.kernel,
        out_type=jax.ShapeDtypeStruct((num_indices, d_model), table.dtype),
        mesh=mesh,
    )
    def gather_kernel(table_hbm, idx_hbm, out_hbm):
        def body(i_vmem, o_vmem):
            pltpu.sync_copy(table_hbm.at[i_vmem.at[0]], o_vmem)

        pltpu.emit_pipeline(
            body,
            grid=(num_indices // _WINDOW,),
            in_specs=[pl.BlockSpec((1, _WINDOW), index_map=lambda i: (0, i))],
            out_specs=[pl.BlockSpec((_WINDOW, d_model), index_map=lambda i: (i, 0))],
            core_axis_name=("core", "subcore"),
            dimension_semantics=(pltpu.PARALLEL,),
        )(idx_hbm, out_hbm)

    return gather_kernel(table, idx_flat)


def kernel(indices, embedding):
    b, s = indices.shape
    v, d = embedding.shape
    flat = indices.reshape(1, b * s).astype(jnp.int32)
    out = _sc_gather(embedding, flat, b * s, d)
    return out.reshape(b, s, d)


# speedup vs baseline: 7.7000x; 1.1810x over previous
import functools

import jax
import jax.numpy as jnp
from jax.experimental import pallas as pl
from jax.experimental.pallas import tpu as pltpu
from jax.experimental.pallas import tpu_sc as plsc

_GW = 128
_KPW = 2
_WINDOW = _GW * _KPW


@functools.partial(jax.jit, static_argnums=(2, 3))
def _sc_gather(table, idx_flat, num_indices, d_model):
    mesh = plsc.VectorSubcoreMesh(core_axis_name="core", subcore_axis_name="subcore")

    @functools.partial(
        pl.kernel,
        out_type=jax.ShapeDtypeStruct((num_indices, d_model), table.dtype),
        mesh=mesh,
        scratch_types=[pltpu.SemaphoreType.DMA],
    )
    def gather_kernel(table_hbm, idx_hbm, out_hbm, sem):
        def body(i_vmem, o_vmem):
            copies = [
                pltpu.async_copy(
                    table_hbm.at[i_vmem.at[j]],
                    o_vmem.at[pl.ds(j * _GW, _GW)],
                    sem,
                )
                for j in range(_KPW)
            ]
            for c in copies:
                c.wait()

        pltpu.emit_pipeline(
            body,
            grid=(num_indices // _WINDOW,),
            in_specs=[pl.BlockSpec((_KPW, _GW), index_map=lambda i: (i, 0))],
            out_specs=[pl.BlockSpec((_WINDOW, d_model), index_map=lambda i: (i, 0))],
            core_axis_name=("core", "subcore"),
            dimension_semantics=(pltpu.PARALLEL,),
        )(idx_hbm, out_hbm)

    return gather_kernel(table, idx_flat)


def kernel(indices, embedding):
    b, s = indices.shape
    v, d = embedding.shape
    flat = indices.reshape(-1, _GW).astype(jnp.int32)
    out = _sc_gather(embedding, flat, b * s, d)
    return out.reshape(b, s, d)
